# baseline (speedup 1.0000x reference)
import functools

import jax
import jax.numpy as jnp
from jax import lax
from jax.experimental import pallas as pl
from jax.experimental.pallas import tpu as pltpu
from jax.experimental.pallas import tpu_sc as plsc

N = 10000
E = 320000
D_IN = 128
HID = 64

NC = 2
NS = 16
NW = NC * NS
EPW = E // NW
C = 80
NCHUNK = EPW // C
NROWCH = N // C


NBUF = 4


def _make_edge_scatter(with_deg):
    grp = HID // 16
    mesh = plsc.VectorSubcoreMesh(core_axis_name="c", subcore_axis_name="s")

    out_types = [jax.ShapeDtypeStruct((NC, N, HID), jnp.float32)]
    scratch = [
        pltpu.VMEM_SHARED((N, HID), jnp.float32),
        pltpu.VMEM((NCHUNK, C), jnp.int32),
        pltpu.VMEM((NCHUNK, C), jnp.int32),
        pltpu.VMEM((NCHUNK, C), jnp.float32),
        [pltpu.VMEM((C, HID), jnp.float32)] * NBUF,
        [pltpu.SemaphoreType.DMA] * NBUF,
        [pltpu.SemaphoreType.DMA] * NBUF,
    ]
    if with_deg:
        out_types.append(jax.ShapeDtypeStruct((NC, N, 16), jnp.float32))
        scratch += [
            pltpu.VMEM_SHARED((N, 16), jnp.float32),
            [pltpu.VMEM((C, 16), jnp.float32)] * NBUF,
            [pltpu.SemaphoreType.DMA] * NBUF,
        ]

    @functools.partial(
        pl.kernel,
        out_type=tuple(out_types),
        mesh=mesh,
        scratch_types=scratch,
        compiler_params=pltpu.CompilerParams(use_tc_tiling_on_sc=False),
    )
    def kern(table, src, dst, ew, *rest):
        if with_deg:
            (out, outd, acc, src_v, dst_v, ew_v, rows, gsem, ssem,
             accd, drows, dsem) = rest
        else:
            out, acc, src_v, dst_v, ew_v, rows, gsem, ssem = rest
        c = lax.axis_index("c")
        s = lax.axis_index("s")
        wid = c * NS + s

        sl = pl.ds(wid * NCHUNK, NCHUNK)
        pltpu.sync_copy(src.at[sl], src_v)
        pltpu.sync_copy(dst.at[sl], dst_v)
        pltpu.sync_copy(ew.at[sl], ew_v)

        zeros = jnp.zeros((16,), jnp.float32)

        def zrow(i, carry):
            for g in range(grp):
                rows[0][i, pl.ds(g * 16, 16)] = zeros
            if with_deg:
                drows[0][i, :] = zeros
            return carry

        lax.fori_loop(0, C, zrow, 0)
        for t in range(-(-NROWCH // NS)):
            j = t * NS + s

            @pl.when(j < NROWCH)
            def _():
                pltpu.sync_copy(rows[0], acc.at[pl.ds(j * C, C)])
                if with_deg:
                    pltpu.sync_copy(drows[0], accd.at[pl.ds(j * C, C)])

        plsc.subcore_barrier()

        def gather(j, b):
            pltpu.async_copy(table.at[src_v.at[j]], rows[b], gsem[b])

        def gather_wait(j, b):
            pltpu.make_async_copy(table.at[src_v.at[j]], rows[b],
                                  gsem[b]).wait()

        def scatter(j, b):
            pltpu.async_copy(rows[b], acc.at[dst_v.at[j]], ssem[b],
                             add=True)
            if with_deg:
                pltpu.async_copy(drows[b], accd.at[dst_v.at[j]], dsem[b],
                                 add=True)

        def scatter_wait(j, b):
            pltpu.make_async_copy(rows[b], acc.at[dst_v.at[j]],
                                  ssem[b]).wait()
            if with_deg:
                pltpu.make_async_copy(drows[b], accd.at[dst_v.at[j]],
                                      dsem[b]).wait()

        def scale(j, b):
            def grp16(g16, carry):
                w16 = ew_v[j, pl.ds(g16 * 16, 16)]
                for l in range(16):
                    e = g16 * 16 + l
                    w = jnp.broadcast_to(w16[l], (16,))
                    for g in range(grp):
                        rows[b][e, pl.ds(g * 16, 16)] = (
                            rows[b][e, pl.ds(g * 16, 16)] * w)
                    if with_deg:
                        drows[b][e, :] = w
                return carry

            lax.fori_loop(0, C // 16, grp16, 0)

        def step(j, b):
            nb = (b + 3) % NBUF
            nj = j + 3

            @pl.when(nj < NCHUNK)
            def _():
                @pl.when(j >= 1)
                def _():
                    scatter_wait(j - 1, nb)

                gather(nj, nb)

            gather_wait(j, b)
            scale(j, b)
            scatter(j, b)

        for b in range(3):
            gather(jnp.int32(b), b)

        def quad(t, carry):
            for b in range(NBUF):
                step(t * NBUF + b, b)
            return carry

        lax.fori_loop(0, NCHUNK // NBUF, quad, 0)
        for b in range(NCHUNK % NBUF):
            step(jnp.int32((NCHUNK // NBUF) * NBUF + b), b)
        for b in range(NBUF):
            j_last = NCHUNK - NBUF + ((b - NCHUNK) % NBUF)
            scatter_wait(jnp.int32(j_last), b)

        plsc.subcore_barrier()
        for t in range(-(-NROWCH // NS)):
            j = t * NS + s

            @pl.when(j < NROWCH)
            def _():
                sl = pl.ds(j * C, C)
                pltpu.sync_copy(acc.at[sl], out.at[c, sl])
                if with_deg:
                    pltpu.sync_copy(accd.at[sl], outd.at[c, sl])

    return kern


_scatter_deg = _make_edge_scatter(True)
_scatter_plain = _make_edge_scatter(False)


def _tc1_body(x_ref, we_ref, be_ref, w1_ref, b1_ref, w2_ref, w3_ref, b3_ref,
              ap_ref, bb_ref, cc_ref):
    h = jnp.dot(x_ref[...], we_ref[...],
                preferred_element_type=jnp.float32) + be_ref[...]
    ap_ref[...] = jnp.dot(h, w1_ref[...],
                          preferred_element_type=jnp.float32) + b1_ref[...]
    bb_ref[...] = jnp.dot(h, w2_ref[...], preferred_element_type=jnp.float32)
    cc_ref[...] = jnp.dot(h, w3_ref[...],
                          preferred_element_type=jnp.float32) + b3_ref[...]


def _tc2_body(p_ref, dp_ref, bb_ref, cc_ref, w1_ref, b1_ref, w2_ref, w3_ref,
              b3_ref, a_ref, bbs_ref, c1_ref):
    agg = p_ref[0] + p_ref[1]
    degw = (dp_ref[0] + dp_ref[1])[:, :1]
    h = jnp.maximum(agg - bb_ref[...] * degw + cc_ref[...], 0.0)
    a_ref[...] = jnp.dot(h, w1_ref[...],
                         preferred_element_type=jnp.float32) + b1_ref[...]
    bbs_ref[...] = jnp.dot(h, w2_ref[...],
                           preferred_element_type=jnp.float32) * degw
    c1_ref[...] = jnp.dot(h, w3_ref[...],
                          preferred_element_type=jnp.float32) + b3_ref[...]


def _tc3_body(p_ref, bbs_ref, cc_ref, out_ref):
    tot = p_ref[0] + p_ref[1]
    out_ref[...] = jnp.maximum(tot - bbs_ref[...] + cc_ref[...], 0.0)


def kernel(x, edge_index, edge_attr, batch, W_emb, b_emb,
           W1_0, b1_0, W2_0, W3_0, b3_0,
           W1_1, b1_1, W2_1, W3_1, b3_1):
    del batch
    src = edge_index[0].reshape(E // C, C)
    dst = edge_index[1].reshape(E // C, C)
    ew2 = edge_attr.reshape(E // C, C)

    f32 = jnp.float32
    ap, bb0, c0 = pl.pallas_call(
        _tc1_body,
        out_shape=(
            jax.ShapeDtypeStruct((N, HID), f32),
            jax.ShapeDtypeStruct((N, HID), f32),
            jax.ShapeDtypeStruct((N, HID), f32),
        ),
    )(x, W_emb, b_emb.reshape(1, HID), W1_0, b1_0.reshape(1, HID),
      W2_0, W3_0, b3_0.reshape(1, HID))

    p0, d0 = _scatter_deg(ap, src, dst, ew2)

    a1, bb1s, c1 = pl.pallas_call(
        _tc2_body,
        out_shape=(
            jax.ShapeDtypeStruct((N, HID), f32),
            jax.ShapeDtypeStruct((N, HID), f32),
            jax.ShapeDtypeStruct((N, HID), f32),
        ),
    )(p0, d0, bb0, c0, W1_1, b1_1.reshape(1, HID), W2_1, W3_1,
      b3_1.reshape(1, HID))

    p1 = _scatter_plain(a1, src, dst, ew2)[0]

    h2 = pl.pallas_call(
        _tc3_body,
        out_shape=jax.ShapeDtypeStruct((N, HID), f32),
    )(p1, bb1s, c1)
    return h2


# speedup vs baseline: 1.9643x; 1.9643x over previous
import functools

import jax
import jax.numpy as jnp
from jax import lax
from jax.experimental import pallas as pl
from jax.experimental.pallas import tpu as pltpu
from jax.experimental.pallas import tpu_sc as plsc

N = 10000
E = 320000
D_IN = 128
HID = 64
W = HID + 16

NC = 2
NS = 16
NW = NC * NS
EPW = E // NW
C = 80
NCHUNK = EPW // C
NROWCH = N // C
NBUF = 4


def _make_edge_scatter():
    grp = W // 16
    mesh = plsc.VectorSubcoreMesh(core_axis_name="c", subcore_axis_name="s")

    @functools.partial(
        pl.kernel,
        out_type=jax.ShapeDtypeStruct((NC, N, W), jnp.float32),
        mesh=mesh,
        scratch_types=[
            pltpu.VMEM_SHARED((N, W), jnp.float32),
            pltpu.VMEM((NCHUNK, C), jnp.int32),
            pltpu.VMEM((NCHUNK, C), jnp.int32),
            pltpu.VMEM((NCHUNK, C), jnp.float32),
            [pltpu.VMEM((C, W), jnp.float32)] * NBUF,
            [pltpu.SemaphoreType.DMA] * NBUF,
            [pltpu.SemaphoreType.DMA] * NBUF,
        ],
        compiler_params=pltpu.CompilerParams(use_tc_tiling_on_sc=False),
    )
    def kern(table, src, dst, ew, out, acc, src_v, dst_v, ew_v, rows,
             gsem, ssem):
        c = lax.axis_index("c")
        s = lax.axis_index("s")
        wid = c * NS + s

        sl = pl.ds(wid * NCHUNK, NCHUNK)
        pltpu.sync_copy(src.at[sl], src_v)
        pltpu.sync_copy(dst.at[sl], dst_v)
        pltpu.sync_copy(ew.at[sl], ew_v)

        zeros = jnp.zeros((16,), jnp.float32)

        def zrow(i, carry):
            for g in range(grp):
                rows[0][i, pl.ds(g * 16, 16)] = zeros
            return carry

        lax.fori_loop(0, C, zrow, 0)
        for t in range(-(-NROWCH // NS)):
            j = t * NS + s

            @pl.when(j < NROWCH)
            def _():
                pltpu.sync_copy(rows[0], acc.at[pl.ds(j * C, C)])

        plsc.subcore_barrier()

        def gather(j, b):
            pltpu.async_copy(table.at[src_v.at[j]], rows[b], gsem[b])

        def gather_wait(j, b):
            pltpu.make_async_copy(table.at[src_v.at[j]], rows[b],
                                  gsem[b]).wait()

        def scatter(j, b):
            pltpu.async_copy(rows[b], acc.at[dst_v.at[j]], ssem[b],
                             add=True)

        def scatter_wait(j, b):
            pltpu.make_async_copy(rows[b], acc.at[dst_v.at[j]],
                                  ssem[b]).wait()

        def scale(j, b):
            def grp16(g16, carry):
                w16 = ew_v[j, pl.ds(g16 * 16, 16)]
                for l in range(16):
                    e = g16 * 16 + l
                    w = jnp.broadcast_to(w16[l], (16,))
                    for g in range(grp):
                        rows[b][e, pl.ds(g * 16, 16)] = (
                            rows[b][e, pl.ds(g * 16, 16)] * w)
                return carry

            lax.fori_loop(0, C // 16, grp16, 0)

        def step(j, b):
            nb = (b + 3) % NBUF
            nj = j + 3

            @pl.when(nj < NCHUNK)
            def _():
                @pl.when(j >= 1)
                def _():
                    scatter_wait(j - 1, nb)

                gather(nj, nb)

            gather_wait(j, b)
            scale(j, b)
            scatter(j, b)

        for b in range(3):
            gather(jnp.int32(b), b)

        def quad(t, carry):
            for b in range(NBUF):
                step(t * NBUF + b, b)
            return carry

        lax.fori_loop(0, NCHUNK // NBUF, quad, 0)
        for b in range(NCHUNK % NBUF):
            step(jnp.int32((NCHUNK // NBUF) * NBUF + b), b)
        for b in range(NBUF):
            j_last = NCHUNK - NBUF + ((b - NCHUNK) % NBUF)
            scatter_wait(jnp.int32(j_last), b)

        plsc.subcore_barrier()
        for t in range(-(-NROWCH // NS)):
            j = t * NS + s

            @pl.when(j < NROWCH)
            def _():
                sl = pl.ds(j * C, C)
                pltpu.sync_copy(acc.at[sl], out.at[c, sl])

    return kern


_edge_scatter_l1 = _make_edge_scatter()
_edge_scatter_l2 = _make_edge_scatter()


def _pad_ones(a):
    return jnp.concatenate([a, jnp.ones((a.shape[0], W - HID), jnp.float32)],
                           axis=1)


def _tc1_body(x_ref, we_ref, be_ref, w1_ref, b1_ref, w2_ref, w3_ref, b3_ref,
              ap_ref, bb_ref, cc_ref):
    h = jnp.dot(x_ref[...], we_ref[...],
                preferred_element_type=jnp.float32) + be_ref[...]
    a = jnp.dot(h, w1_ref[...], preferred_element_type=jnp.float32) + b1_ref[...]
    ap_ref[...] = _pad_ones(a)
    bb_ref[...] = jnp.dot(h, w2_ref[...], preferred_element_type=jnp.float32)
    cc_ref[...] = jnp.dot(h, w3_ref[...],
                          preferred_element_type=jnp.float32) + b3_ref[...]


def _tc2_body(p_ref, bb_ref, cc_ref, w1_ref, b1_ref, w2_ref, w3_ref, b3_ref,
              ap_ref, bbs_ref, c1_ref):
    tot = p_ref[0] + p_ref[1]
    agg = tot[:, :HID]
    degw = tot[:, HID:HID + 1]
    h = jnp.maximum(agg - bb_ref[...] * degw + cc_ref[...], 0.0)
    a = jnp.dot(h, w1_ref[...], preferred_element_type=jnp.float32) + b1_ref[...]
    ap_ref[...] = _pad_ones(a)
    bbs_ref[...] = jnp.dot(h, w2_ref[...],
                           preferred_element_type=jnp.float32) * degw
    c1_ref[...] = jnp.dot(h, w3_ref[...],
                          preferred_element_type=jnp.float32) + b3_ref[...]


def _tc3_body(p_ref, bbs_ref, cc_ref, out_ref):
    tot = p_ref[0] + p_ref[1]
    out_ref[...] = jnp.maximum(tot[:, :HID] - bbs_ref[...] + cc_ref[...], 0.0)


def kernel(x, edge_index, edge_attr, batch, W_emb, b_emb,
           W1_0, b1_0, W2_0, W3_0, b3_0,
           W1_1, b1_1, W2_1, W3_1, b3_1):
    del batch
    src = edge_index[0].reshape(E // C, C)
    dst = edge_index[1].reshape(E // C, C)
    ew2 = edge_attr.reshape(E // C, C)

    f32 = jnp.float32
    ap, bb0, c0 = pl.pallas_call(
        _tc1_body,
        out_shape=(
            jax.ShapeDtypeStruct((N, W), f32),
            jax.ShapeDtypeStruct((N, HID), f32),
            jax.ShapeDtypeStruct((N, HID), f32),
        ),
    )(x, W_emb, b_emb.reshape(1, HID), W1_0, b1_0.reshape(1, HID),
      W2_0, W3_0, b3_0.reshape(1, HID))

    p0 = _edge_scatter_l1(ap, src, dst, ew2)

    a1p, bb1s, c1 = pl.pallas_call(
        _tc2_body,
        out_shape=(
            jax.ShapeDtypeStruct((N, W), f32),
            jax.ShapeDtypeStruct((N, HID), f32),
            jax.ShapeDtypeStruct((N, HID), f32),
        ),
    )(p0, bb0, c0, W1_1, b1_1.reshape(1, HID), W2_1, W3_1,
      b3_1.reshape(1, HID))

    p1 = _edge_scatter_l2(a1p, src, dst, ew2)

    h2 = pl.pallas_call(
        _tc3_body,
        out_shape=jax.ShapeDtypeStruct((N, HID), f32),
    )(p1, bb1s, c1)
    return h2
